# baseline (speedup 1.0000x reference)
import functools

import jax
import jax.numpy as jnp
from jax import lax
from jax.experimental import pallas as pl
from jax.experimental.pallas import tpu as pltpu
from jax.experimental.pallas import tpu_sc as plsc

NC = 2
NS = 16
NW = NC * NS
L = 16
CH = 64
G = 4
BLK = 512


def _round_up(v, m):
    return (v + m - 1) // m * m


def _deg_body(rows_hbm, out_hbm, idx_v, acc_v, red_v, sum_v, shared, sem,
              *, per_w, n_pad):
    c = lax.axis_index("c")
    s = lax.axis_index("s")
    rpt = n_pad // NS

    zero = jnp.zeros((L,), jnp.float32)

    def zbody(i, _):
        acc_v[pl.ds(i * L, L)] = zero
        return ()

    lax.fori_loop(0, n_pad // L, zbody, ())

    wid = c * NS + s
    pltpu.sync_copy(rows_hbm.at[pl.ds(wid * per_w, per_w)], idx_v)
    ones = jnp.ones((L,), jnp.float32)

    def sbody(k, _):
        idx = idx_v[pl.ds(k * L, L)]
        plsc.addupdate_scatter(acc_v, [idx], ones)
        return ()

    lax.fori_loop(0, per_w // L, sbody, ())

    pltpu.sync_copy(acc_v, shared.at[s])
    plsc.subcore_barrier()
    for t in range(NS):
        pltpu.sync_copy(shared.at[t, pl.ds(s * rpt, rpt)], red_v.at[t])

    def rbody(j, _):
        tot = red_v[0, pl.ds(j * L, L)]
        for t in range(1, NS):
            tot = tot + red_v[t, pl.ds(j * L, L)]
        sum_v[pl.ds(j * L, L)] = tot
        return ()

    lax.fori_loop(0, rpt // L, rbody, ())
    pltpu.sync_copy(sum_v, out_hbm.at[c, pl.ds(s * rpt, rpt)])


NB = 3
GS = 80
S0 = 2
S1 = 2


def _spmm_body(f_hbm, cols_hbm, rows_hbm, zeros_hbm, out_hbm,
               cidx, ridx, buf, acc, gsem, ssem, *, n_pad):
    c = lax.axis_index("c")
    s = lax.axis_index("s")
    rpt = n_pad // NS
    pltpu.sync_copy(zeros_hbm, acc.at[pl.ds(s * rpt, rpt)])
    plsc.subcore_barrier()

    def fire_gather(j, k):
        pltpu.async_copy(f_hbm.at[cidx.at[j]], buf.at[k], gsem)

    def drain_gather(k):
        pltpu.make_async_copy(f_hbm.at[cidx.at[0]], buf.at[k], gsem).wait()

    def fire_scatter(j, k):
        pltpu.async_copy(buf.at[k], acc.at[ridx.at[j]], ssem, add=True)

    def drain_scatter(k):
        pltpu.make_async_copy(f_hbm.at[cidx.at[0]], buf.at[k], ssem).wait()

    base0 = s * (S0 * GS)
    base1 = NS * (S0 * GS) + s * (S1 * GS)
    wbase = jnp.where(c == 0, base0, base1)
    nstage = jnp.where(c == 0, S0, S1)

    for h in range(max(S0, S1)):
        @pl.when(h < nstage)
        def _():
            base = wbase + h * GS
            pltpu.sync_copy(cols_hbm.at[pl.ds(base, GS)], cidx)
            pltpu.sync_copy(rows_hbm.at[pl.ds(base, GS)], ridx)
            for k in range(NB):
                fire_gather(k, k)

            def body(j, _):
                r = lax.rem(j, NB)
                for k in range(NB):
                    @pl.when(r == k)
                    def _():
                        drain_gather(k)
                        fire_scatter(j, k)
                        nxt = j + NB

                        @pl.when(nxt < GS)
                        def _():
                            drain_scatter(k)
                            fire_gather(nxt, k)
                return ()

            lax.fori_loop(0, GS, body, (), unroll=False)
            for _ in range(NB):
                drain_scatter(0)

    plsc.subcore_barrier()
    sl = pl.ds(s * rpt, rpt)
    pltpu.sync_copy(acc.at[sl], out_hbm.at[c, sl])


def _proj_body(x_ref, theta_ref, degp_ref, f_ref):
    deg = degp_ref[0] + degp_ref[1]
    norm = lax.rsqrt(deg + 1e-6)
    f = jnp.dot(x_ref[...], theta_ref[...], preferred_element_type=jnp.float32)
    f_ref[...] = f * norm


def _out_body(x_ref, wh_ref, wt_ref, bt_ref, p_ref, degp_ref, o_ref):
    deg = degp_ref[0] + degp_ref[1]
    norm = lax.rsqrt(deg + 1e-6)
    xs = x_ref[...]
    fh = (p_ref[0] + p_ref[1]) * norm
    f_het = jnp.dot(xs, wh_ref[...], preferred_element_type=jnp.float32)
    gate = jax.nn.sigmoid(
        jnp.dot(xs, wt_ref[...], preferred_element_type=jnp.float32)
        + bt_ref[...])
    o = gate * fh + (1.0 - gate) * f_het
    o_ref[...] = jnp.where(o > 0.0, o, jnp.exp(jnp.minimum(o, 0.0)) - 1.0)


def kernel(x, edge_index, W_t, b_t, W_h, theta):
    n = x.shape[1]
    d = x.shape[2]
    e = edge_index.shape[1]

    n_pad = _round_up(n + 1, BLK)
    e_pad = NS * (S0 + S1) * GS * CH
    assert e <= e_pad
    per_w = e_pad // NW
    rpt = n_pad // NS

    xs = x[0]
    x_pad = jnp.pad(xs, ((0, n_pad - n), (0, 0)))
    rows = edge_index[0]
    cols = edge_index[1]
    pad_e = e_pad - e
    rows_p = jnp.concatenate([rows, jnp.full((pad_e,), n, dtype=jnp.int32)])
    cols_p = jnp.concatenate([cols, jnp.zeros((pad_e,), dtype=jnp.int32)])
    rows2 = rows_p.reshape(e_pad // CH, CH)
    cols2 = cols_p.reshape(e_pad // CH, CH)

    zerosd = jnp.zeros((rpt, d), dtype=jnp.float32)

    mesh = plsc.VectorSubcoreMesh(core_axis_name="c", subcore_axis_name="s")

    deg_call = pl.kernel(
        functools.partial(_deg_body, per_w=per_w, n_pad=n_pad),
        out_type=jax.ShapeDtypeStruct((NC, n_pad), jnp.float32),
        mesh=mesh,
        compiler_params=pltpu.CompilerParams(needs_layout_passes=False),
        scratch_types=[
            pltpu.VMEM((per_w,), jnp.int32),
            pltpu.VMEM((n_pad,), jnp.float32),
            pltpu.VMEM((NS, rpt), jnp.float32),
            pltpu.VMEM((rpt,), jnp.float32),
            pltpu.VMEM_SHARED((NS, n_pad), jnp.float32),
            pltpu.SemaphoreType.DMA,
        ],
    )
    degp = deg_call(rows_p)
    degp3 = degp.reshape(NC, n_pad, 1)

    grid = n_pad // BLK
    f = pl.pallas_call(
        _proj_body,
        grid=(grid,),
        in_specs=[
            pl.BlockSpec((BLK, d), lambda i: (i, 0)),
            pl.BlockSpec((d, d), lambda i: (0, 0)),
            pl.BlockSpec((NC, BLK, 1), lambda i: (0, i, 0)),
        ],
        out_specs=pl.BlockSpec((BLK, d), lambda i: (i, 0)),
        out_shape=jax.ShapeDtypeStruct((n_pad, d), jnp.float32),
    )(x_pad, theta, degp3)

    spmm_call = pl.kernel(
        functools.partial(_spmm_body, n_pad=n_pad),
        out_type=jax.ShapeDtypeStruct((NC, n_pad, d), jnp.float32),
        mesh=mesh,
        scratch_types=[
            pltpu.VMEM((GS, CH), jnp.int32),
            pltpu.VMEM((GS, CH), jnp.int32),
            pltpu.VMEM((NB, CH, d), jnp.float32),
            pltpu.VMEM_SHARED((n_pad, d), jnp.float32),
            pltpu.SemaphoreType.DMA,
            pltpu.SemaphoreType.DMA,
        ],
    )
    p = spmm_call(f, cols2, rows2, zerosd)

    out = pl.pallas_call(
        _out_body,
        grid=(grid,),
        in_specs=[
            pl.BlockSpec((BLK, d), lambda i: (i, 0)),
            pl.BlockSpec((d, d), lambda i: (0, 0)),
            pl.BlockSpec((d, d), lambda i: (0, 0)),
            pl.BlockSpec((1, d), lambda i: (0, 0)),
            pl.BlockSpec((NC, BLK, d), lambda i: (0, i, 0)),
            pl.BlockSpec((NC, BLK, 1), lambda i: (0, i, 0)),
        ],
        out_specs=pl.BlockSpec((BLK, d), lambda i: (i, 0)),
        out_shape=jax.ShapeDtypeStruct((n_pad, d), jnp.float32),
    )(x_pad, W_h, W_t, b_t.reshape(1, d), p, degp3)

    return out[:n][None]


# speedup vs baseline: 1.0238x; 1.0238x over previous
import functools

import jax
import jax.numpy as jnp
from jax import lax
from jax.experimental import pallas as pl
from jax.experimental.pallas import tpu as pltpu
from jax.experimental.pallas import tpu_sc as plsc

NC = 2
NS = 16
NW = NC * NS
L = 16
CH = 128
G = 4
BLK = 512


def _round_up(v, m):
    return (v + m - 1) // m * m


def _deg_body(rows_hbm, out_hbm, idx_v, acc_v, red_v, sum_v, shared, sem,
              *, per_w, n_pad):
    c = lax.axis_index("c")
    s = lax.axis_index("s")
    rpt = n_pad // NS

    zero = jnp.zeros((L,), jnp.float32)

    def zbody(i, _):
        acc_v[pl.ds(i * L, L)] = zero
        return ()

    lax.fori_loop(0, n_pad // L, zbody, ())

    wid = c * NS + s
    pltpu.sync_copy(rows_hbm.at[pl.ds(wid * per_w, per_w)], idx_v)
    ones = jnp.ones((L,), jnp.float32)

    def sbody(k, _):
        idx = idx_v[pl.ds(k * L, L)]
        plsc.addupdate_scatter(acc_v, [idx], ones)
        return ()

    lax.fori_loop(0, per_w // L, sbody, ())

    pltpu.sync_copy(acc_v, shared.at[s])
    plsc.subcore_barrier()
    for t in range(NS):
        pltpu.sync_copy(shared.at[t, pl.ds(s * rpt, rpt)], red_v.at[t])

    def rbody(j, _):
        tot = red_v[0, pl.ds(j * L, L)]
        for t in range(1, NS):
            tot = tot + red_v[t, pl.ds(j * L, L)]
        sum_v[pl.ds(j * L, L)] = tot
        return ()

    lax.fori_loop(0, rpt // L, rbody, ())
    pltpu.sync_copy(sum_v, out_hbm.at[c, pl.ds(s * rpt, rpt)])


NB = 2
GS = 40
S0 = 2
S1 = 2


def _spmm_body(f_hbm, cols_hbm, rows_hbm, zeros_hbm, out_hbm,
               cidx, ridx, buf, acc, gsem, ssem, *, n_pad):
    c = lax.axis_index("c")
    s = lax.axis_index("s")
    rpt = n_pad // NS
    pltpu.sync_copy(zeros_hbm, acc.at[pl.ds(s * rpt, rpt)])
    plsc.subcore_barrier()

    def fire_gather(j, k):
        pltpu.async_copy(f_hbm.at[cidx.at[j]], buf.at[k], gsem)

    def drain_gather(k):
        pltpu.make_async_copy(f_hbm.at[cidx.at[0]], buf.at[k], gsem).wait()

    def fire_scatter(j, k):
        pltpu.async_copy(buf.at[k], acc.at[ridx.at[j]], ssem, add=True)

    def drain_scatter(k):
        pltpu.make_async_copy(f_hbm.at[cidx.at[0]], buf.at[k], ssem).wait()

    base0 = s * (S0 * GS)
    base1 = NS * (S0 * GS) + s * (S1 * GS)
    wbase = jnp.where(c == 0, base0, base1)
    nstage = jnp.where(c == 0, S0, S1)

    for h in range(max(S0, S1)):
        @pl.when(h < nstage)
        def _():
            base = wbase + h * GS
            pltpu.sync_copy(cols_hbm.at[pl.ds(base, GS)], cidx)
            pltpu.sync_copy(rows_hbm.at[pl.ds(base, GS)], ridx)
            for k in range(NB):
                fire_gather(k, k)

            def body(j, _):
                r = lax.rem(j, NB)
                for k in range(NB):
                    @pl.when(r == k)
                    def _():
                        drain_gather(k)
                        fire_scatter(j, k)
                        nxt = j + NB

                        @pl.when(nxt < GS)
                        def _():
                            drain_scatter(k)
                            fire_gather(nxt, k)
                return ()

            lax.fori_loop(0, GS, body, (), unroll=False)
            for _ in range(NB):
                drain_scatter(0)

    plsc.subcore_barrier()
    sl = pl.ds(s * rpt, rpt)
    pltpu.sync_copy(acc.at[sl], out_hbm.at[c, sl])


def _proj_body(x_ref, theta_ref, degp_ref, f_ref):
    deg = degp_ref[0] + degp_ref[1]
    norm = lax.rsqrt(deg + 1e-6)
    f = jnp.dot(x_ref[...], theta_ref[...], preferred_element_type=jnp.float32)
    f_ref[...] = f * norm


def _out_body(x_ref, wh_ref, wt_ref, bt_ref, p_ref, degp_ref, o_ref):
    deg = degp_ref[0] + degp_ref[1]
    norm = lax.rsqrt(deg + 1e-6)
    xs = x_ref[...]
    fh = (p_ref[0] + p_ref[1]) * norm
    f_het = jnp.dot(xs, wh_ref[...], preferred_element_type=jnp.float32)
    gate = jax.nn.sigmoid(
        jnp.dot(xs, wt_ref[...], preferred_element_type=jnp.float32)
        + bt_ref[...])
    o = gate * fh + (1.0 - gate) * f_het
    o_ref[...] = jnp.where(o > 0.0, o, jnp.exp(jnp.minimum(o, 0.0)) - 1.0)


def kernel(x, edge_index, W_t, b_t, W_h, theta):
    n = x.shape[1]
    d = x.shape[2]
    e = edge_index.shape[1]

    n_pad = _round_up(n + 1, BLK)
    e_pad = NS * (S0 + S1) * GS * CH
    assert e <= e_pad
    per_w = e_pad // NW
    rpt = n_pad // NS

    xs = x[0]
    x_pad = jnp.pad(xs, ((0, n_pad - n), (0, 0)))
    rows = edge_index[0]
    cols = edge_index[1]
    pad_e = e_pad - e
    rows_p = jnp.concatenate([rows, jnp.full((pad_e,), n, dtype=jnp.int32)])
    cols_p = jnp.concatenate([cols, jnp.zeros((pad_e,), dtype=jnp.int32)])
    rows2 = rows_p.reshape(e_pad // CH, CH)
    cols2 = cols_p.reshape(e_pad // CH, CH)

    zerosd = jnp.zeros((rpt, d), dtype=jnp.float32)

    mesh = plsc.VectorSubcoreMesh(core_axis_name="c", subcore_axis_name="s")

    deg_call = pl.kernel(
        functools.partial(_deg_body, per_w=per_w, n_pad=n_pad),
        out_type=jax.ShapeDtypeStruct((NC, n_pad), jnp.float32),
        mesh=mesh,
        compiler_params=pltpu.CompilerParams(needs_layout_passes=False),
        scratch_types=[
            pltpu.VMEM((per_w,), jnp.int32),
            pltpu.VMEM((n_pad,), jnp.float32),
            pltpu.VMEM((NS, rpt), jnp.float32),
            pltpu.VMEM((rpt,), jnp.float32),
            pltpu.VMEM_SHARED((NS, n_pad), jnp.float32),
            pltpu.SemaphoreType.DMA,
        ],
    )
    degp = deg_call(rows_p)
    degp3 = degp.reshape(NC, n_pad, 1)

    grid = n_pad // BLK
    f = pl.pallas_call(
        _proj_body,
        grid=(grid,),
        in_specs=[
            pl.BlockSpec((BLK, d), lambda i: (i, 0)),
            pl.BlockSpec((d, d), lambda i: (0, 0)),
            pl.BlockSpec((NC, BLK, 1), lambda i: (0, i, 0)),
        ],
        out_specs=pl.BlockSpec((BLK, d), lambda i: (i, 0)),
        out_shape=jax.ShapeDtypeStruct((n_pad, d), jnp.float32),
    )(x_pad, theta, degp3)

    spmm_call = pl.kernel(
        functools.partial(_spmm_body, n_pad=n_pad),
        out_type=jax.ShapeDtypeStruct((NC, n_pad, d), jnp.float32),
        mesh=mesh,
        scratch_types=[
            pltpu.VMEM((GS, CH), jnp.int32),
            pltpu.VMEM((GS, CH), jnp.int32),
            pltpu.VMEM((NB, CH, d), jnp.float32),
            pltpu.VMEM_SHARED((n_pad, d), jnp.float32),
            pltpu.SemaphoreType.DMA,
            pltpu.SemaphoreType.DMA,
        ],
    )
    p = spmm_call(f, cols2, rows2, zerosd)

    out = pl.pallas_call(
        _out_body,
        grid=(grid,),
        in_specs=[
            pl.BlockSpec((BLK, d), lambda i: (i, 0)),
            pl.BlockSpec((d, d), lambda i: (0, 0)),
            pl.BlockSpec((d, d), lambda i: (0, 0)),
            pl.BlockSpec((1, d), lambda i: (0, 0)),
            pl.BlockSpec((NC, BLK, d), lambda i: (0, i, 0)),
            pl.BlockSpec((NC, BLK, 1), lambda i: (0, i, 0)),
        ],
        out_specs=pl.BlockSpec((BLK, d), lambda i: (i, 0)),
        out_shape=jax.ShapeDtypeStruct((n_pad, d), jnp.float32),
    )(x_pad, W_h, W_t, b_t.reshape(1, d), p, degp3)

    return out[:n][None]


# speedup vs baseline: 1.0335x; 1.0095x over previous
import functools

import jax
import jax.numpy as jnp
from jax import lax
from jax.experimental import pallas as pl
from jax.experimental.pallas import tpu as pltpu
from jax.experimental.pallas import tpu_sc as plsc

NC = 2
NS = 16
NW = NC * NS
L = 16
CH = 128
G = 4
BLK = 512


def _round_up(v, m):
    return (v + m - 1) // m * m


def _deg_body(rows_hbm, out_hbm, idx_v, acc_v, red_v, sum_v, shared, sem,
              *, per_w, n_pad):
    c = lax.axis_index("c")
    s = lax.axis_index("s")
    rpt = n_pad // NS

    zero = jnp.zeros((L,), jnp.float32)

    def zbody(i, _):
        acc_v[pl.ds(i * L, L)] = zero
        return ()

    lax.fori_loop(0, n_pad // L, zbody, ())

    wid = c * NS + s
    pltpu.sync_copy(rows_hbm.at[pl.ds(wid * per_w, per_w)], idx_v)
    ones = jnp.ones((L,), jnp.float32)

    def sbody(k, _):
        idx = idx_v[pl.ds(k * L, L)]
        plsc.addupdate_scatter(acc_v, [idx], ones)
        return ()

    lax.fori_loop(0, per_w // L, sbody, ())

    pltpu.sync_copy(acc_v, shared.at[s])
    plsc.subcore_barrier()
    for t in range(NS):
        pltpu.sync_copy(shared.at[t, pl.ds(s * rpt, rpt)], red_v.at[t])

    def rbody(j, _):
        tot = red_v[0, pl.ds(j * L, L)]
        for t in range(1, NS):
            tot = tot + red_v[t, pl.ds(j * L, L)]
        sum_v[pl.ds(j * L, L)] = tot
        return ()

    lax.fori_loop(0, rpt // L, rbody, ())
    pltpu.sync_copy(sum_v, out_hbm.at[c, pl.ds(s * rpt, rpt)])


NB = 2
GS = 40
S0 = 2
S1 = 2


def _spmm_body(f_hbm, cols_hbm, rows_hbm, out_hbm,
               cidx, ridx, buf, acc, gsem, ssem, *, n_pad):
    c = lax.axis_index("c")
    s = lax.axis_index("s")
    rpt = n_pad // NS

    zero = jnp.zeros((L,), jnp.float32)

    def zbody(i, _):
        for q in range(128 // L):
            buf[0, i, pl.ds(q * L, L)] = zero
        return ()

    lax.fori_loop(0, CH, zbody, ())
    for b in range(rpt // CH):
        pltpu.sync_copy(buf.at[0], acc.at[pl.ds(s * rpt + b * CH, CH)])
    plsc.subcore_barrier()

    def fire_gather(j, k):
        pltpu.async_copy(f_hbm.at[cidx.at[j]], buf.at[k], gsem)

    def drain_gather(k):
        pltpu.make_async_copy(f_hbm.at[cidx.at[0]], buf.at[k], gsem).wait()

    def fire_scatter(j, k):
        pltpu.async_copy(buf.at[k], acc.at[ridx.at[j]], ssem, add=True)

    def drain_scatter(k):
        pltpu.make_async_copy(f_hbm.at[cidx.at[0]], buf.at[k], ssem).wait()

    base0 = s * (S0 * GS)
    base1 = NS * (S0 * GS) + s * (S1 * GS)
    wbase = jnp.where(c == 0, base0, base1)
    nstage = jnp.where(c == 0, S0, S1)

    for h in range(max(S0, S1)):
        @pl.when(h < nstage)
        def _():
            base = wbase + h * GS
            pltpu.sync_copy(cols_hbm.at[pl.ds(base, GS)], cidx)
            pltpu.sync_copy(rows_hbm.at[pl.ds(base, GS)], ridx)
            for k in range(NB):
                fire_gather(k, k)

            def body(j, _):
                r = lax.rem(j, NB)
                for k in range(NB):
                    @pl.when(r == k)
                    def _():
                        drain_gather(k)
                        fire_scatter(j, k)
                        nxt = j + NB

                        @pl.when(nxt < GS)
                        def _():
                            drain_scatter(k)
                            fire_gather(nxt, k)
                return ()

            lax.fori_loop(0, GS, body, (), unroll=False)
            for _ in range(NB):
                drain_scatter(0)

    plsc.subcore_barrier()
    sl = pl.ds(s * rpt, rpt)
    pltpu.sync_copy(acc.at[sl], out_hbm.at[c, sl])


def _proj_body(x_ref, theta_ref, degp_ref, f_ref):
    deg = degp_ref[0] + degp_ref[1]
    norm = lax.rsqrt(deg + 1e-6)
    f = jnp.dot(x_ref[...], theta_ref[...], preferred_element_type=jnp.float32)
    f_ref[...] = f * norm


def _out_body(x_ref, wh_ref, wt_ref, bt_ref, p_ref, degp_ref, o_ref):
    deg = degp_ref[0] + degp_ref[1]
    norm = lax.rsqrt(deg + 1e-6)
    xs = x_ref[...]
    fh = (p_ref[0] + p_ref[1]) * norm
    f_het = jnp.dot(xs, wh_ref[...], preferred_element_type=jnp.float32)
    gate = jax.nn.sigmoid(
        jnp.dot(xs, wt_ref[...], preferred_element_type=jnp.float32)
        + bt_ref[...])
    o = gate * fh + (1.0 - gate) * f_het
    o_ref[...] = jnp.where(o > 0.0, o, jnp.exp(jnp.minimum(o, 0.0)) - 1.0)


def kernel(x, edge_index, W_t, b_t, W_h, theta):
    n = x.shape[1]
    d = x.shape[2]
    e = edge_index.shape[1]

    n_pad = _round_up(n + 1, BLK)
    e_pad = NS * (S0 + S1) * GS * CH
    assert e <= e_pad
    per_w = e_pad // NW
    rpt = n_pad // NS

    xs = x[0]
    x_pad = jnp.pad(xs, ((0, n_pad - n), (0, 0)))
    rows = edge_index[0]
    cols = edge_index[1]
    pad_e = e_pad - e
    rows_p = jnp.concatenate([rows, jnp.full((pad_e,), n, dtype=jnp.int32)])
    cols_p = jnp.concatenate([cols, jnp.zeros((pad_e,), dtype=jnp.int32)])
    rows2 = rows_p.reshape(e_pad // CH, CH)
    cols2 = cols_p.reshape(e_pad // CH, CH)

    mesh = plsc.VectorSubcoreMesh(core_axis_name="c", subcore_axis_name="s")

    deg_call = pl.kernel(
        functools.partial(_deg_body, per_w=per_w, n_pad=n_pad),
        out_type=jax.ShapeDtypeStruct((NC, n_pad), jnp.float32),
        mesh=mesh,
        compiler_params=pltpu.CompilerParams(needs_layout_passes=False),
        scratch_types=[
            pltpu.VMEM((per_w,), jnp.int32),
            pltpu.VMEM((n_pad,), jnp.float32),
            pltpu.VMEM((NS, rpt), jnp.float32),
            pltpu.VMEM((rpt,), jnp.float32),
            pltpu.VMEM_SHARED((NS, n_pad), jnp.float32),
            pltpu.SemaphoreType.DMA,
        ],
    )
    degp = deg_call(rows_p)
    degp3 = degp.reshape(NC, n_pad, 1)

    grid = n_pad // BLK
    f = pl.pallas_call(
        _proj_body,
        grid=(grid,),
        in_specs=[
            pl.BlockSpec((BLK, d), lambda i: (i, 0)),
            pl.BlockSpec((d, d), lambda i: (0, 0)),
            pl.BlockSpec((NC, BLK, 1), lambda i: (0, i, 0)),
        ],
        out_specs=pl.BlockSpec((BLK, d), lambda i: (i, 0)),
        out_shape=jax.ShapeDtypeStruct((n_pad, d), jnp.float32),
    )(x_pad, theta, degp3)

    spmm_call = pl.kernel(
        functools.partial(_spmm_body, n_pad=n_pad),
        out_type=jax.ShapeDtypeStruct((NC, n_pad, d), jnp.float32),
        mesh=mesh,
        scratch_types=[
            pltpu.VMEM((GS, CH), jnp.int32),
            pltpu.VMEM((GS, CH), jnp.int32),
            pltpu.VMEM((NB, CH, d), jnp.float32),
            pltpu.VMEM_SHARED((n_pad, d), jnp.float32),
            pltpu.SemaphoreType.DMA,
            pltpu.SemaphoreType.DMA,
        ],
    )
    p = spmm_call(f, cols2, rows2)

    out = pl.pallas_call(
        _out_body,
        grid=(grid,),
        in_specs=[
            pl.BlockSpec((BLK, d), lambda i: (i, 0)),
            pl.BlockSpec((d, d), lambda i: (0, 0)),
            pl.BlockSpec((d, d), lambda i: (0, 0)),
            pl.BlockSpec((1, d), lambda i: (0, 0)),
            pl.BlockSpec((NC, BLK, d), lambda i: (0, i, 0)),
            pl.BlockSpec((NC, BLK, 1), lambda i: (0, i, 0)),
        ],
        out_specs=pl.BlockSpec((BLK, d), lambda i: (i, 0)),
        out_shape=jax.ShapeDtypeStruct((n_pad, d), jnp.float32),
    )(x_pad, W_h, W_t, b_t.reshape(1, d), p, degp3)

    return out[:n][None]
